# baseline (speedup 1.0000x reference)
import functools

import jax
import jax.numpy as jnp
from jax import lax
from jax.experimental import pallas as pl
from jax.experimental.pallas import tpu as pltpu
from jax.experimental.pallas import tpu_sc as plsc

B = 4
N = 5000
C = 256
L = 16
NC = 2
NS = 16
NW = NC * NS
CHUNK = 80
WPB = NW // B
NSC = WPB * CHUNK
NTC = N - NSC
RSC = B * NSC
GROUPS = CHUNK // L
KSTEPS = C // L
PSTRIDE = L + 1
BR = 872
GJ = NTC // BR + 1
BOX_THRESHOLD = 0.2


def _tree_max(vals):
    vals = list(vals)
    while len(vals) > 1:
        nxt = [jnp.maximum(a, b) for a, b in zip(vals[::2], vals[1::2])]
        if len(vals) % 2:
            nxt.append(vals[-1])
        vals = nxt
    return vals[0]


def _sc_body(logits_hbm, boxes_hbm, ts_hbm, dets_hbm, keep_hbm,
             lbuf, bbuf, tsbuf, partial, dbuf, kbuf):
    wid = lax.axis_index("s") * NC + lax.axis_index("c")
    bi = wid // WPB
    rs = NTC + (wid % WPB) * CHUNK
    pltpu.sync_copy(ts_hbm, tsbuf)
    lanes = lax.iota(jnp.int32, L)

    pltpu.sync_copy(logits_hbm.at[bi, pl.ds(rs, CHUNK), :], lbuf)
    pltpu.sync_copy(boxes_hbm.at[pl.ds((bi * N + rs) * 4, CHUNK * 4)], bbuf)

    @plsc.parallel_loop(0, CHUNK, step=1, unroll=8)
    def row_body(r):
        acc = _tree_max([lbuf[r, pl.ds(k * L, L)] for k in range(KSTEPS)])
        partial[pl.ds(r * PSTRIDE, L)] = acc

    bvec = jnp.full((L,), 0, jnp.int32) + bi
    img_h = plsc.load_gather(tsbuf, [bvec * 2]).astype(jnp.float32)
    img_w = plsc.load_gather(tsbuf, [bvec * 2 + 1]).astype(jnp.float32)

    for g in range(GROUPS):
        rows = g * L + lanes
        m = _tree_max([plsc.load_gather(partial, [rows * PSTRIDE + j])
                       for j in range(L)])
        scores = 1.0 / (1.0 + jnp.exp(-m))

        cx = plsc.load_gather(bbuf, [rows * 4])
        cy = plsc.load_gather(bbuf, [rows * 4 + 1])
        w = plsc.load_gather(bbuf, [rows * 4 + 2])
        h = plsc.load_gather(bbuf, [rows * 4 + 3])

        keep = scores > BOX_THRESHOLD
        kf = jnp.where(keep, 1.0, 0.0)
        x0 = (cx - 0.5 * w) * img_w * kf
        y0 = (cy - 0.5 * h) * img_h * kf
        x1 = (cx + 0.5 * w) * img_w * kf
        y1 = (cy + 0.5 * h) * img_h * kf
        sm = scores * kf

        plsc.store_scatter(dbuf, [rows * 5], sm)
        plsc.store_scatter(dbuf, [rows * 5 + 1], x0)
        plsc.store_scatter(dbuf, [rows * 5 + 2], y0)
        plsc.store_scatter(dbuf, [rows * 5 + 3], x1)
        plsc.store_scatter(dbuf, [rows * 5 + 4], y1)
        kbuf[pl.ds(g * L, L)] = keep.astype(jnp.int32)

    pltpu.sync_copy(dbuf, dets_hbm.at[pl.ds(wid * CHUNK * 5, CHUNK * 5)])
    pltpu.sync_copy(kbuf, keep_hbm.at[pl.ds(wid * CHUNK, CHUNK)])


def _run_sc(logits3d, boxes_flat, ts_flat):
    mesh = plsc.VectorSubcoreMesh(core_axis_name="c", subcore_axis_name="s")
    run = functools.partial(
        pl.kernel,
        out_type=[
            jax.ShapeDtypeStruct((RSC * 5,), jnp.float32),
            jax.ShapeDtypeStruct((RSC,), jnp.int32),
        ],
        mesh=mesh,
        compiler_params=pltpu.CompilerParams(needs_layout_passes=False),
        scratch_types=[
            pltpu.VMEM((CHUNK, C), jnp.float32),
            pltpu.VMEM((CHUNK * 4,), jnp.float32),
            pltpu.VMEM((B * 2,), jnp.int32),
            pltpu.VMEM((CHUNK * PSTRIDE,), jnp.float32),
            pltpu.VMEM((CHUNK * 5,), jnp.float32),
            pltpu.VMEM((CHUNK,), jnp.int32),
        ],
    )(_sc_body)
    return run(logits3d, boxes_flat, ts_flat)


def _tc_body(ts_ref, logits_ref, boxes_ref, dets_ref, keep_ref):
    bi = pl.program_id(0)
    x = logits_ref[...]
    m2 = jnp.maximum(x[:, :, :C // 2], x[:, :, C // 2:])
    m = jnp.max(m2, axis=-1, keepdims=True)
    scores = 1.0 / (1.0 + jnp.exp(-m))

    hsel = [jnp.float32(ts_ref[0, 2 * i]) for i in range(B)]
    wsel = [jnp.float32(ts_ref[0, 2 * i + 1]) for i in range(B)]
    img_h = jnp.where(bi == 0, hsel[0],
                      jnp.where(bi == 1, hsel[1],
                                jnp.where(bi == 2, hsel[2], hsel[3])))
    img_w = jnp.where(bi == 0, wsel[0],
                      jnp.where(bi == 1, wsel[1],
                                jnp.where(bi == 2, wsel[2], wsel[3])))

    bx = boxes_ref[...]
    cx, cy = bx[:, :, 0:1], bx[:, :, 1:2]
    w, h = bx[:, :, 2:3], bx[:, :, 3:4]

    keep = scores > BOX_THRESHOLD
    kf = jnp.where(keep, 1.0, 0.0)
    x0 = (cx - 0.5 * w) * img_w * kf
    y0 = (cy - 0.5 * h) * img_h * kf
    x1 = (cx + 0.5 * w) * img_w * kf
    y1 = (cy + 0.5 * h) * img_h * kf
    sm = scores * kf

    dets_ref[...] = jnp.concatenate([sm, x0, y0, x1, y1], axis=2)
    keep_ref[...] = keep.astype(jnp.int32)


def _run_tc(logits3d, boxes3d, ts2d):
    return pl.pallas_call(
        _tc_body,
        grid=(B, GJ),
        in_specs=[
            pl.BlockSpec((1, 2 * B), lambda b, j: (0, 0)),
            pl.BlockSpec((1, BR, C), lambda b, j: (b, jnp.minimum(j, GJ - 2),
                                                   0)),
            pl.BlockSpec((1, BR, 4), lambda b, j: (b, jnp.minimum(j, GJ - 2),
                                                   0)),
        ],
        out_specs=[
            pl.BlockSpec((1, BR, 5), lambda b, j: (b, j, 0)),
            pl.BlockSpec((1, BR, 1), lambda b, j: (b, j, 0)),
        ],
        out_shape=[
            jax.ShapeDtypeStruct((B, N, 5), jnp.float32),
            jax.ShapeDtypeStruct((B, N, 1), jnp.int32),
        ],
    )(ts2d, logits3d, boxes3d)


@jax.jit
def _post_process(logits3d, boxes3d, ts):
    dets_sc_flat, keep_sc = _run_sc(
        logits3d, boxes3d.reshape(B * N * 4), ts.reshape(B * 2))
    dets_tc, keep_tc = _run_tc(logits3d, boxes3d, ts.reshape(1, B * 2))
    dets = lax.dynamic_update_slice(
        dets_tc, dets_sc_flat.reshape(B, NSC, 5), (0, NTC, 0))
    keep = lax.dynamic_update_slice(
        keep_tc, keep_sc.reshape(B, NSC, 1), (0, NTC, 0))
    return dets, keep.reshape(B, N)


def kernel(pred_logits, pred_boxes, target_sizes):
    dets, keep_i = _post_process(pred_logits, pred_boxes, target_sizes)
    return dets, keep_i.astype(jnp.bool_)


# speedup vs baseline: 1.1313x; 1.1313x over previous
import functools

import jax
import jax.numpy as jnp
from jax import lax
from jax.experimental import pallas as pl
from jax.experimental.pallas import tpu as pltpu
from jax.experimental.pallas import tpu_sc as plsc

B = 4
N = 5000
C = 256
R = B * N
L = 16
NC = 2
NS = 16
NW = NC * NS
CHUNK = 160
CPW = 4
GROUPS = CHUNK // L
KSTEPS = C // L
PSTRIDE = L + 1
BOX_THRESHOLD = 0.2


def _tree_max(vals):
    vals = list(vals)
    while len(vals) > 1:
        nxt = [jnp.maximum(a, b) for a, b in zip(vals[::2], vals[1::2])]
        if len(vals) % 2:
            nxt.append(vals[-1])
        vals = nxt
    return vals[0]


def _sc_body(logits_hbm, boxes_hbm, ts_hbm, dets_hbm, keep_hbm,
             lbuf0, lbuf1, bbuf0, bbuf1, tsbuf, partial,
             dbuf0, dbuf1, kbuf0, kbuf1,
             lsem0, lsem1, bsem0, bsem1, osem0, osem1):
    lbufs = (lbuf0, lbuf1)
    bbufs = (bbuf0, bbuf1)
    dbufs = (dbuf0, dbuf1)
    kbufs = (kbuf0, kbuf1)
    lsems = (lsem0, lsem1)
    bsems = (bsem0, bsem1)
    osems = (osem0, osem1)

    wid = lax.axis_index("s") * NC + lax.axis_index("c")
    pltpu.sync_copy(ts_hbm, tsbuf)
    lanes = lax.iota(jnp.int32, L)

    bases = [jnp.minimum(wid * (CPW * CHUNK) + c * CHUNK, R - CHUNK)
             for c in range(CPW)]

    def start_in(c):
        p = c % 2
        return (
            pltpu.async_copy(
                logits_hbm.at[pl.ds(bases[c], CHUNK), :], lbufs[p], lsems[p]),
            pltpu.async_copy(
                boxes_hbm.at[pl.ds(bases[c] * 4, CHUNK * 4)], bbufs[p],
                bsems[p]),
        )

    in_descs = {0: start_in(0), 1: start_in(1)}
    out_descs = {}

    for c in range(CPW):
        p = c % 2
        base = bases[c]
        lbuf, bbuf, dbuf, kbuf = lbufs[p], bbufs[p], dbufs[p], kbufs[p]
        for d in in_descs.pop(c):
            d.wait()
        if c >= 2:
            for d in out_descs.pop(c - 2):
                d.wait()


        @plsc.parallel_loop(0, CHUNK, step=1, unroll=8)
        def row_body(r):
            acc = _tree_max([lbuf[r, pl.ds(k * L, L)] for k in range(KSTEPS)])
            partial[pl.ds(r * PSTRIDE, L)] = acc

        for g in range(GROUPS):
            rows = g * L + lanes
            m = _tree_max([plsc.load_gather(partial, [rows * PSTRIDE + j])
                           for j in range(L)])
            scores = 1.0 / (1.0 + jnp.exp(-m))

            grow = base + rows
            b = ((grow >= N).astype(jnp.int32)
                 + (grow >= 2 * N).astype(jnp.int32)
                 + (grow >= 3 * N).astype(jnp.int32))
            img_h = plsc.load_gather(tsbuf, [b * 2]).astype(jnp.float32)
            img_w = plsc.load_gather(tsbuf, [b * 2 + 1]).astype(jnp.float32)

            cx = plsc.load_gather(bbuf, [rows * 4])
            cy = plsc.load_gather(bbuf, [rows * 4 + 1])
            w = plsc.load_gather(bbuf, [rows * 4 + 2])
            h = plsc.load_gather(bbuf, [rows * 4 + 3])

            keep = scores > BOX_THRESHOLD
            kf = jnp.where(keep, 1.0, 0.0)
            x0 = (cx - 0.5 * w) * img_w * kf
            y0 = (cy - 0.5 * h) * img_h * kf
            x1 = (cx + 0.5 * w) * img_w * kf
            y1 = (cy + 0.5 * h) * img_h * kf
            sm = scores * kf

            plsc.store_scatter(dbuf, [rows * 5], sm)
            plsc.store_scatter(dbuf, [rows * 5 + 1], x0)
            plsc.store_scatter(dbuf, [rows * 5 + 2], y0)
            plsc.store_scatter(dbuf, [rows * 5 + 3], x1)
            plsc.store_scatter(dbuf, [rows * 5 + 4], y1)
            kbuf[pl.ds(g * L, L)] = keep.astype(jnp.int32)

        out_descs[c] = (
            pltpu.async_copy(dbuf, dets_hbm.at[pl.ds(base * 5, CHUNK * 5)],
                             osems[p]),
            pltpu.async_copy(kbuf, keep_hbm.at[pl.ds(base, CHUNK)], osems[p]),
        )
        if c + 2 < CPW:
            in_descs[c + 2] = start_in(c + 2)

    for c in (CPW - 2, CPW - 1):
        for d in out_descs.pop(c):
            d.wait()


@jax.jit
def _post_process_sc(logits2d, boxes_flat, ts_flat):
    mesh = plsc.VectorSubcoreMesh(core_axis_name="c", subcore_axis_name="s")
    run = functools.partial(
        pl.kernel,
        out_type=[
            jax.ShapeDtypeStruct((R * 5,), jnp.float32),
            jax.ShapeDtypeStruct((R,), jnp.int32),
        ],
        mesh=mesh,
        compiler_params=pltpu.CompilerParams(needs_layout_passes=False),
        scratch_types=[
            pltpu.VMEM((CHUNK, C), jnp.float32),
            pltpu.VMEM((CHUNK, C), jnp.float32),
            pltpu.VMEM((CHUNK * 4,), jnp.float32),
            pltpu.VMEM((CHUNK * 4,), jnp.float32),
            pltpu.VMEM((B * 2,), jnp.int32),
            pltpu.VMEM((CHUNK * PSTRIDE,), jnp.float32),
            pltpu.VMEM((CHUNK * 5,), jnp.float32),
            pltpu.VMEM((CHUNK * 5,), jnp.float32),
            pltpu.VMEM((CHUNK,), jnp.int32),
            pltpu.VMEM((CHUNK,), jnp.int32),
            pltpu.SemaphoreType.DMA,
            pltpu.SemaphoreType.DMA,
            pltpu.SemaphoreType.DMA,
            pltpu.SemaphoreType.DMA,
            pltpu.SemaphoreType.DMA,
            pltpu.SemaphoreType.DMA,
        ],
    )(_sc_body)
    return run(logits2d, boxes_flat, ts_flat)


def kernel(pred_logits, pred_boxes, target_sizes):
    logits2d = pred_logits.reshape(R, C)
    boxes_flat = pred_boxes.reshape(R * 4)
    ts_flat = target_sizes.reshape(B * 2)
    dets_flat, keep_i = _post_process_sc(logits2d, boxes_flat, ts_flat)
    dets = dets_flat.reshape(B, N, 5)
    keep = keep_i.reshape(B, N).astype(jnp.bool_)
    return dets, keep
